# Initial kernel scaffold; baseline (speedup 1.0000x reference)
import jax
import jax.numpy as jnp
from jax.experimental import pallas as pl


def kernel(x, edge_index, target_nodes, W1, b1, W2, b2, Wih, Whh, bih, bhh, C1w, C1b, C2w, C2b):
    raise NotImplementedError("write your pallas kernel here")


# speedup vs baseline: 62.7029x; 62.7029x over previous
import functools
import jax
import jax.numpy as jnp
from jax import lax
from jax.experimental import pallas as pl
from jax.experimental.pallas import tpu as pltpu
from jax.experimental.pallas import tpu_sc as plsc

NN = 10000
EE = 320000
TT = 3
DD = 128
HH = 64
NPAD = 10240
NTILE = 16
CPT = NPAD // NTILE
NV_C = CPT // 16
EPT1 = 20480
NV_E = EPT1 // 16
EP1 = NTILE * EPT1
CH = 128
EPT2 = 10112
NCH = EPT2 // CH
EPAD = 32 * EPT2
NEG = -1e30

_mesh = plsc.VectorSubcoreMesh(core_axis_name="c", subcore_axis_name="s")
_sc_params = pltpu.CompilerParams(needs_layout_passes=False,
                                  use_tc_tiling_on_sc=False)


def _zero_vmem(ref, n, dtype):
  def body(i, _):
    ref[pl.ds(i * 16, 16)] = jnp.zeros((16,), dtype)
    return 0
  lax.fori_loop(0, n // 16, body, 0)


def _reduce_slots(red_v, dtype):
  def body(v, _):
    acc = red_v[0, pl.ds(v * 16, 16)]
    for k in range(1, NTILE):
      acc = acc + red_v[k, pl.ds(v * 16, 16)]
    return acc
  return body


def _bfs_body(es, ed, initmask, masks_out, degs_out,
              s_v, d_v, mask_v, cnt_v, red_v, stage_sh, mask_sh):
  cid = lax.axis_index("c")
  sid = lax.axis_index("s")

  @pl.when(cid == 0)
  def _():
    for t in range(TT):
      pltpu.sync_copy(es.at[t, sid, 0], s_v)
      pltpu.sync_copy(ed.at[t, sid, 0], d_v)
      pltpu.sync_copy(initmask, mask_v)
      for _hop in range(2):
        _zero_vmem(cnt_v, NPAD, jnp.int32)

        def ebody(j, _):
          s16 = s_v[pl.ds(j * 16, 16)]
          d16 = d_v[pl.ds(j * 16, 16)]
          ms = plsc.load_gather(mask_v, [s16])
          md = plsc.load_gather(mask_v, [d16])
          plsc.addupdate_scatter(cnt_v, [d16], ms)
          plsc.addupdate_scatter(cnt_v, [s16], md)
          return 0
        lax.fori_loop(0, NV_E, ebody, 0)

        pltpu.sync_copy(cnt_v, stage_sh.at[sid, 0])
        plsc.subcore_barrier()
        pltpu.sync_copy(stage_sh.at[:, 0, pl.ds(sid * CPT, CPT)], red_v)
        radd = _reduce_slots(red_v, jnp.int32)

        def rbody(v, _):
          acc = radd(v, None)
          old = mask_v[pl.ds(sid * CPT + v * 16, 16)]
          mask_v[pl.ds(sid * CPT + v * 16, 16)] = jnp.where(acc > 0, 1, old)
          return 0
        lax.fori_loop(0, NV_C, rbody, 0)

        pltpu.sync_copy(mask_v.at[pl.ds(sid * CPT, CPT)],
                        mask_sh.at[pl.ds(sid * CPT, CPT)])
        plsc.subcore_barrier()
        pltpu.sync_copy(mask_sh, mask_v)
        plsc.subcore_barrier()

      pltpu.sync_copy(mask_v.at[pl.ds(sid * CPT, CPT)],
                      masks_out.at[t, 0, pl.ds(sid * CPT, CPT)])

      _zero_vmem(cnt_v, NPAD, jnp.int32)

      def dbody(j, _):
        s16 = s_v[pl.ds(j * 16, 16)]
        d16 = d_v[pl.ds(j * 16, 16)]
        ms = plsc.load_gather(mask_v, [s16])
        plsc.addupdate_scatter(cnt_v, [d16], ms)
        return 0
      lax.fori_loop(0, NV_E, dbody, 0)

      pltpu.sync_copy(cnt_v, stage_sh.at[sid, 0])
      plsc.subcore_barrier()
      pltpu.sync_copy(stage_sh.at[:, 0, pl.ds(sid * CPT, CPT)], red_v)
      radd = _reduce_slots(red_v, jnp.int32)

      def dbody2(v, _):
        cnt_v[pl.ds(sid * CPT + v * 16, 16)] = radd(v, None)
        return 0
      lax.fori_loop(0, NV_C, dbody2, 0)
      pltpu.sync_copy(cnt_v.at[pl.ds(sid * CPT, CPT)],
                      degs_out.at[t, 0, pl.ds(sid * CPT, CPT)])
      plsc.subcore_barrier()


_bfs_call = pl.kernel(
    _bfs_body,
    out_type=(jax.ShapeDtypeStruct((TT, 1, NPAD), jnp.int32),
              jax.ShapeDtypeStruct((TT, 1, NPAD), jnp.int32)),
    mesh=_mesh,
    compiler_params=_sc_params,
    scratch_types=[
        pltpu.VMEM((EPT1,), jnp.int32),
        pltpu.VMEM((EPT1,), jnp.int32),
        pltpu.VMEM((NPAD,), jnp.int32),
        pltpu.VMEM((NPAD,), jnp.int32),
        pltpu.VMEM((NTILE, CPT), jnp.int32),
        pltpu.VMEM_SHARED((NTILE, 1, NPAD), jnp.int32),
        pltpu.VMEM_SHARED((NPAD,), jnp.int32),
    ],
)


def _msg_body(sidx_hbm, didx_hbm, xwdm_hbm, zrows_hbm, apart_out,
              sidx_v, didx_v, rows0, rows1, sem0, sem1, a_sh):
  cid = lax.axis_index("c")
  sid = lax.axis_index("s")
  wid = cid * NTILE + sid
  rows = [rows0, rows1]
  sems = [sem0, sem1]
  for t in range(TT):
    pltpu.sync_copy(zrows_hbm, a_sh.at[pl.ds(sid * CPT, CPT)])
    pltpu.sync_copy(sidx_hbm.at[t, wid], sidx_v)
    pltpu.sync_copy(didx_hbm.at[t, wid], didx_v)
    plsc.subcore_barrier()
    cp = pltpu.async_copy(xwdm_hbm.at[sidx_v.at[0]], rows0, sem0)
    for j in range(NCH):
      if j + 1 < NCH:
        cpn = pltpu.async_copy(
            xwdm_hbm.at[sidx_v.at[j + 1]],
            rows[(j + 1) % 2], sems[(j + 1) % 2])
      cp.wait()
      pltpu.sync_copy(rows[j % 2], a_sh.at[didx_v.at[j]], add=True)
      if j + 1 < NCH:
        cp = cpn
    plsc.subcore_barrier()
    pltpu.sync_copy(a_sh.at[pl.ds(sid * CPT, CPT)],
                    apart_out.at[cid, t, pl.ds(sid * CPT, CPT)])
    plsc.subcore_barrier()


_msg_call = pl.kernel(
    _msg_body,
    out_type=jax.ShapeDtypeStruct((2, TT, NPAD, HH), jnp.float32),
    mesh=_mesh,
    compiler_params=_sc_params,
    scratch_types=[
        pltpu.VMEM((NCH, CH), jnp.int32),
        pltpu.VMEM((NCH, CH), jnp.int32),
        pltpu.VMEM((CH, HH), jnp.float32),
        pltpu.VMEM((CH, HH), jnp.float32),
        pltpu.SemaphoreType.DMA,
        pltpu.SemaphoreType.DMA,
        pltpu.VMEM_SHARED((NPAD, HH), jnp.float32),
    ],
)


def _scal_body(es, ed, gdm_hbm, acc_out,
               s_v, d_v, val_v, acc_v, red_v, stage_sh):
  cid = lax.axis_index("c")
  sid = lax.axis_index("s")

  @pl.when(cid == 0)
  def _():
    for t in range(TT):
      pltpu.sync_copy(es.at[t, sid, 0], s_v)
      pltpu.sync_copy(ed.at[t, sid, 0], d_v)
      pltpu.sync_copy(gdm_hbm.at[t, 0], val_v)
      _zero_vmem(acc_v, NPAD, jnp.float32)

      def ebody(j, _):
        s16 = s_v[pl.ds(j * 16, 16)]
        d16 = d_v[pl.ds(j * 16, 16)]
        gs = plsc.load_gather(val_v, [s16])
        plsc.addupdate_scatter(acc_v, [d16], gs)
        return 0
      lax.fori_loop(0, NV_E, ebody, 0)

      pltpu.sync_copy(acc_v, stage_sh.at[sid, 0])
      plsc.subcore_barrier()
      pltpu.sync_copy(stage_sh.at[:, 0, pl.ds(sid * CPT, CPT)], red_v)
      radd = _reduce_slots(red_v, jnp.float32)

      def rbody(v, _):
        acc_v[pl.ds(sid * CPT + v * 16, 16)] = radd(v, None)
        return 0
      lax.fori_loop(0, NV_C, rbody, 0)
      pltpu.sync_copy(acc_v.at[pl.ds(sid * CPT, CPT)],
                      acc_out.at[t, 0, pl.ds(sid * CPT, CPT)])
      plsc.subcore_barrier()


_scal_call = pl.kernel(
    _scal_body,
    out_type=jax.ShapeDtypeStruct((TT, 1, NPAD), jnp.float32),
    mesh=_mesh,
    compiler_params=_sc_params,
    scratch_types=[
        pltpu.VMEM((EPT1,), jnp.int32),
        pltpu.VMEM((EPT1,), jnp.int32),
        pltpu.VMEM((NPAD,), jnp.float32),
        pltpu.VMEM((NPAD,), jnp.float32),
        pltpu.VMEM((NTILE, CPT), jnp.float32),
        pltpu.VMEM_SHARED((NTILE, 1, NPAD), jnp.float32),
    ],
)


def _win_body(es, ed, nstar_hbm, ccnt_out,
              s_v, d_v, nb_v, cnt_v, red_v, stage_sh):
  cid = lax.axis_index("c")
  sid = lax.axis_index("s")

  @pl.when(cid == 0)
  def _():
    for t in range(TT):
      pltpu.sync_copy(es.at[t, sid, 0], s_v)
      pltpu.sync_copy(ed.at[t, sid, 0], d_v)
      pltpu.sync_copy(nstar_hbm.at[t, 0], nb_v)
      _zero_vmem(cnt_v, NPAD, jnp.int32)
      nst16 = nb_v[...]

      def ebody(j, _):
        s16 = s_v[pl.ds(j * 16, 16)]
        d16 = d_v[pl.ds(j * 16, 16)]
        hit = jnp.where(d16 == nst16, 1, 0)
        plsc.addupdate_scatter(cnt_v, [s16], hit)
        return 0
      lax.fori_loop(0, NV_E, ebody, 0)

      pltpu.sync_copy(cnt_v, stage_sh.at[sid, 0])
      plsc.subcore_barrier()
      pltpu.sync_copy(stage_sh.at[:, 0, pl.ds(sid * CPT, CPT)], red_v)
      radd = _reduce_slots(red_v, jnp.int32)

      def rbody(v, _):
        cnt_v[pl.ds(sid * CPT + v * 16, 16)] = radd(v, None)
        return 0
      lax.fori_loop(0, NV_C, rbody, 0)
      pltpu.sync_copy(cnt_v.at[pl.ds(sid * CPT, CPT)],
                      ccnt_out.at[t, 0, pl.ds(sid * CPT, CPT)])
      plsc.subcore_barrier()


_win_call = pl.kernel(
    _win_body,
    out_type=jax.ShapeDtypeStruct((TT, 1, NPAD), jnp.int32),
    mesh=_mesh,
    compiler_params=_sc_params,
    scratch_types=[
        pltpu.VMEM((EPT1,), jnp.int32),
        pltpu.VMEM((EPT1,), jnp.int32),
        pltpu.VMEM((16,), jnp.int32),
        pltpu.VMEM((NPAD,), jnp.int32),
        pltpu.VMEM((NTILE, CPT), jnp.int32),
        pltpu.VMEM_SHARED((NTILE, 1, NPAD), jnp.int32),
    ],
)


def _prep_body(x_ref, w1a_ref, rc_ref, rnc_ref, tgt_ref, masks_ref, degs_ref,
               xw_out, dinv_out, xwdm_out):
  xw = jnp.dot(x_ref[...], w1a_ref[...], preferred_element_type=jnp.float32)
  ii = lax.broadcasted_iota(jnp.int32, (NN, 1), 0)
  center = (ii == tgt_ref[0]) | (ii == tgt_ref[1])
  xw = xw + jnp.where(center, rc_ref[...], rnc_ref[...])
  xw_out[pl.ds(0, NN), :] = xw
  xw_out[pl.ds(NN, NPAD - NN), :] = jnp.zeros((NPAD - NN, HH), jnp.float32)
  for t in range(TT):
    deg = (degs_ref[pl.ds(t, 1), :] + 1).astype(jnp.float32)
    dinv = lax.rsqrt(deg)
    dinv_out[pl.ds(t, 1), :] = dinv
    dm = dinv * masks_ref[pl.ds(t, 1), :].astype(jnp.float32)
    dmcol = jnp.reshape(dm, (NPAD, 1))
    xwdm_out[pl.ds(t * NPAD, NPAD), :] = xw_out[...] * dmcol


def _prep_call(x, w1a, rc, rnc, tgt, masks, degs):
  return pl.pallas_call(
      _prep_body,
      out_shape=(jax.ShapeDtypeStruct((NPAD, HH), jnp.float32),
                 jax.ShapeDtypeStruct((TT, NPAD), jnp.float32),
                 jax.ShapeDtypeStruct((TT * NPAD, HH), jnp.float32)),
      in_specs=[pl.BlockSpec(memory_space=pltpu.VMEM)] * 4
      + [pl.BlockSpec(memory_space=pltpu.SMEM)]
      + [pl.BlockSpec(memory_space=pltpu.VMEM)] * 2,
  )(x, w1a, rc, rnc, tgt, masks, degs)


def _h1_body(ap_ref, xw_ref, dinv_ref, masks_ref, b1_ref, w2c_ref,
             h1_out, g_out, gdm_out):
  for t in range(TT):
    a = ap_ref[0, pl.ds(t * NPAD, NPAD), :] + ap_ref[1, pl.ds(t * NPAD, NPAD), :]
    dinv = dinv_ref[pl.ds(t, 1), :]
    dcol = jnp.reshape(dinv, (NPAD, 1))
    h1 = jnp.maximum(
        a * dcol + xw_ref[...] * (dcol * dcol) + b1_ref[...], 0.0)
    h1_out[pl.ds(t * NPAD, NPAD), :] = h1
    g = lax.dot_general(w2c_ref[...], h1, (((1,), (1,)), ((), ())),
                        preferred_element_type=jnp.float32)
    g_out[pl.ds(t, 1), :] = g
    gdm_out[pl.ds(t, 1), :] = (
        g * dinv * masks_ref[pl.ds(t, 1), :].astype(jnp.float32))


def _h1_call(ap, xw, dinv, masks, b1r, w2cr):
  return pl.pallas_call(
      _h1_body,
      out_shape=(jax.ShapeDtypeStruct((TT * NPAD, HH), jnp.float32),
                 jax.ShapeDtypeStruct((TT, NPAD), jnp.float32),
                 jax.ShapeDtypeStruct((TT, NPAD), jnp.float32)),
  )(ap, xw, dinv, masks, b1r, w2cr)


def _arg_body(acc_ref, g_ref, dinv_ref, masks_ref, b2h_ref,
              nstar_out, dstar_out):
  for t in range(TT):
    dinv = dinv_ref[pl.ds(t, 1), :]
    tp = jnp.maximum(
        dinv * acc_ref[pl.ds(t, 1), :]
        + g_ref[pl.ds(t, 1), :] * dinv * dinv + b2h_ref[0, 0], 0.0)
    key = jnp.where(masks_ref[pl.ds(t, 1), :] > 0, tp, NEG)
    m = jnp.max(key)
    ii = lax.broadcasted_iota(jnp.int32, (1, NPAD), 1)
    nst = jnp.min(jnp.where(key == m, ii, NPAD))
    dl = jnp.max(jnp.where(ii == nst, dinv, 0.0))
    nstar_out[pl.ds(t, 1), :] = jnp.full((1, 16), nst, jnp.int32)
    dstar_out[pl.ds(t, 1), :] = jnp.full((1, 16), dl, jnp.float32)


def _arg_call(acc, g, dinv, masks, b2h):
  return pl.pallas_call(
      _arg_body,
      out_shape=(jax.ShapeDtypeStruct((TT, 16), jnp.int32),
                 jax.ShapeDtypeStruct((TT, 16), jnp.float32)),
      in_specs=[pl.BlockSpec(memory_space=pltpu.VMEM)] * 4
      + [pl.BlockSpec(memory_space=pltpu.SMEM)],
  )(acc, g, dinv, masks, b2h)


def _final_body(ccnt_ref, dinv_ref, masks_ref, h1_ref, nstar_ref, dstar_ref,
                w2_ref, b2_ref, wir_ref, wiz_ref, win_ref,
                whr_ref, whz_ref, whn_ref, bir_ref, biz_ref, bin_ref,
                bhr_ref, bhz_ref, bhn_ref,
                c1w_ref, c1b_ref, c2w_ref, c2b_ref, out_ref):
  h = jnp.zeros((1, HH), jnp.float32)
  for t in range(TT):
    nst = nstar_ref[t, 0]
    dl = dstar_ref[t, 0]
    crow = (ccnt_ref[pl.ds(t, 1), :].astype(jnp.float32)
            * dinv_ref[pl.ds(t, 1), :]
            * masks_ref[pl.ds(t, 1), :].astype(jnp.float32))
    ii = lax.broadcasted_iota(jnp.int32, (1, NPAD), 1)
    oh = (ii == nst).astype(jnp.float32)
    row = dl * crow + (dl * dl) * oh
    agg = lax.dot_general(row, h1_ref[pl.ds(t * NPAD, NPAD), :],
                          (((1,), (0,)), ((), ())),
                          preferred_element_type=jnp.float32)
    emb = jnp.maximum(
        lax.dot_general(agg, w2_ref[...], (((1,), (0,)), ((), ())),
                        preferred_element_type=jnp.float32) + b2_ref[...], 0.0)
    def mm(v, w_ref, b_ref):
      return lax.dot_general(v, w_ref[...], (((1,), (1,)), ((), ())),
                             preferred_element_type=jnp.float32) + b_ref[...]
    r = jax.nn.sigmoid(mm(emb, wir_ref, bir_ref) + mm(h, whr_ref, bhr_ref))
    z = jax.nn.sigmoid(mm(emb, wiz_ref, biz_ref) + mm(h, whz_ref, bhz_ref))
    nn_ = jnp.tanh(mm(emb, win_ref, bin_ref) + r * mm(h, whn_ref, bhn_ref))
    h = (1.0 - z) * nn_ + z * h
  c = jnp.maximum(
      lax.dot_general(h, c1w_ref[...], (((1,), (1,)), ((), ())),
                      preferred_element_type=jnp.float32) + c1b_ref[...], 0.0)
  score = jax.nn.sigmoid(
      jnp.sum(c * c2w_ref[...], axis=1, keepdims=True) + c2b_ref[...])
  out_ref[...] = score


def _final_call(ccnt, dinv, masks, h1, nstar, dstar, w2, b2r,
                wih, whh, bihr, bhhr, c1w, c1br, c2w, c2br):
  gru = []
  for w in (wih, whh):
    gru += [w[:HH], w[HH:2 * HH], w[2 * HH:]]
  for b in (bihr, bhhr):
    gru += [b[:, :HH], b[:, HH:2 * HH], b[:, 2 * HH:]]
  return pl.pallas_call(
      _final_body,
      out_shape=jax.ShapeDtypeStruct((1, 1), jnp.float32),
      in_specs=[pl.BlockSpec(memory_space=pltpu.VMEM)] * 4
      + [pl.BlockSpec(memory_space=pltpu.SMEM)] * 2
      + [pl.BlockSpec(memory_space=pltpu.VMEM)] * 18,
  )(ccnt, dinv, masks, h1, nstar, dstar, w2, b2r,
    *gru, c1w, c1br, c2w, c2br)


@jax.jit
def kernel(x, edge_index, target_nodes, W1, b1, W2, b2,
           Wih, Whh, bih, bhh, C1w, C1b, C2w, C2b):
  ei = edge_index.astype(jnp.int32)
  tgt = target_nodes.astype(jnp.int32)
  initmask = jnp.zeros((NPAD,), jnp.int32).at[tgt].set(1)

  pad1 = EP1 - EE
  es = jnp.pad(ei[:, 0, :], ((0, 0), (0, pad1)),
               constant_values=NPAD - 1).reshape(TT, NTILE, 1, EPT1)
  ed = jnp.pad(ei[:, 1, :], ((0, 0), (0, pad1)),
               constant_values=NPAD - 1).reshape(TT, NTILE, 1, EPT1)

  masks3, degs3 = _bfs_call(es, ed, initmask)
  masks = masks3.reshape(TT, NPAD)
  degs = degs3.reshape(TT, NPAD)

  w1a = W1[:DD]
  rc = W1[DD].reshape(1, HH)
  rnc = W1[DD + 1].reshape(1, HH)
  xw, dinv, xwdm = _prep_call(x, w1a, rc, rnc, tgt, masks, degs)

  pad = EPAD - EE
  spad = jnp.pad(ei[:, 0, :], ((0, 0), (0, pad)), constant_values=NPAD - 1)
  dpad = jnp.pad(ei[:, 1, :], ((0, 0), (0, pad)), constant_values=NPAD - 1)
  toff = (jnp.arange(TT, dtype=jnp.int32) * NPAD)[:, None, None, None]
  sidx = spad.reshape(TT, 32, NCH, CH) + toff
  didx = dpad.reshape(TT, 32, NCH, CH)
  zrows = jnp.zeros((CPT, HH), jnp.float32)
  ap = _msg_call(sidx, didx, xwdm, zrows)

  h1, g, gdm = _h1_call(ap.reshape(2, TT * NPAD, HH), xw, dinv, masks,
                        b1.reshape(1, HH), W2[:, -1].reshape(1, HH))
  acc = _scal_call(es, ed, gdm.reshape(TT, 1, NPAD)).reshape(TT, NPAD)
  nstar, dstar = _arg_call(acc, g, dinv, masks, b2[-1].reshape(1, 1))
  ccnt = _win_call(es, ed, nstar.reshape(TT, 1, 16)).reshape(TT, NPAD)
  out = _final_call(ccnt, dinv, masks, h1, nstar, dstar, W2,
                    b2.reshape(1, HH), Wih, Whh, bih.reshape(1, 3 * HH),
                    bhh.reshape(1, 3 * HH), C1w, C1b.reshape(1, HH // 2),
                    C2w, C2b.reshape(1, 1))
  return out.reshape(())


# speedup vs baseline: 73.6572x; 1.1747x over previous
import functools
import jax
import jax.numpy as jnp
from jax import lax
from jax.experimental import pallas as pl
from jax.experimental.pallas import tpu as pltpu
from jax.experimental.pallas import tpu_sc as plsc

NN = 10000
EE = 320000
TT = 3
DD = 128
HH = 64
NPAD = 10240
NTILE = 16
CPT = NPAD // NTILE
NV_C = CPT // 16
EPT1 = 20480
NV_E = EPT1 // 16
EP1 = NTILE * EPT1
CH = 128
EPT2 = 10112
NCH = EPT2 // CH
EPAD = 32 * EPT2
NEG = -1e30

_mesh = plsc.VectorSubcoreMesh(core_axis_name="c", subcore_axis_name="s")
_sc_params = pltpu.CompilerParams(needs_layout_passes=False,
                                  use_tc_tiling_on_sc=False)


def _zero_vmem(ref, n, dtype):
  def body(i, _):
    ref[pl.ds(i * 16, 16)] = jnp.zeros((16,), dtype)
    return 0
  lax.fori_loop(0, n // 16, body, 0)


def _reduce_slots(red_v, dtype):
  def body(v, _):
    acc = red_v[0, pl.ds(v * 16, 16)]
    for k in range(1, NTILE):
      acc = acc + red_v[k, pl.ds(v * 16, 16)]
    return acc
  return body


def _bfs_body(es, ed, initmask, masks_out, degs_out,
              s_v, d_v, mask_v, cnt_v, red_v, stage_sh, mask_sh):
  cid = lax.axis_index("c")
  sid = lax.axis_index("s")

  for t in range(TT):
    @pl.when(cid == t % 2)
    def _(t=t):
      pltpu.sync_copy(es.at[t, sid, 0], s_v)
      pltpu.sync_copy(ed.at[t, sid, 0], d_v)
      pltpu.sync_copy(initmask, mask_v)
      for _hop in range(2):
        _zero_vmem(cnt_v, NPAD, jnp.int32)

        def ebody(j, _):
          s16 = s_v[pl.ds(j * 16, 16)]
          d16 = d_v[pl.ds(j * 16, 16)]
          ms = plsc.load_gather(mask_v, [s16])
          md = plsc.load_gather(mask_v, [d16])
          plsc.addupdate_scatter(cnt_v, [d16], ms)
          plsc.addupdate_scatter(cnt_v, [s16], md)
          return 0
        lax.fori_loop(0, NV_E, ebody, 0)

        pltpu.sync_copy(cnt_v, stage_sh.at[sid, 0])
        plsc.subcore_barrier()
        pltpu.sync_copy(stage_sh.at[:, 0, pl.ds(sid * CPT, CPT)], red_v)
        radd = _reduce_slots(red_v, jnp.int32)

        def rbody(v, _):
          acc = radd(v, None)
          old = mask_v[pl.ds(sid * CPT + v * 16, 16)]
          mask_v[pl.ds(sid * CPT + v * 16, 16)] = jnp.where(acc > 0, 1, old)
          return 0
        lax.fori_loop(0, NV_C, rbody, 0)

        pltpu.sync_copy(mask_v.at[pl.ds(sid * CPT, CPT)],
                        mask_sh.at[pl.ds(sid * CPT, CPT)])
        plsc.subcore_barrier()
        pltpu.sync_copy(mask_sh, mask_v)
        plsc.subcore_barrier()

      pltpu.sync_copy(mask_v.at[pl.ds(sid * CPT, CPT)],
                      masks_out.at[t, 0, pl.ds(sid * CPT, CPT)])

      _zero_vmem(cnt_v, NPAD, jnp.int32)

      def dbody(j, _):
        s16 = s_v[pl.ds(j * 16, 16)]
        d16 = d_v[pl.ds(j * 16, 16)]
        ms = plsc.load_gather(mask_v, [s16])
        plsc.addupdate_scatter(cnt_v, [d16], ms)
        return 0
      lax.fori_loop(0, NV_E, dbody, 0)

      pltpu.sync_copy(cnt_v, stage_sh.at[sid, 0])
      plsc.subcore_barrier()
      pltpu.sync_copy(stage_sh.at[:, 0, pl.ds(sid * CPT, CPT)], red_v)
      radd = _reduce_slots(red_v, jnp.int32)

      def dbody2(v, _):
        cnt_v[pl.ds(sid * CPT + v * 16, 16)] = radd(v, None)
        return 0
      lax.fori_loop(0, NV_C, dbody2, 0)
      pltpu.sync_copy(cnt_v.at[pl.ds(sid * CPT, CPT)],
                      degs_out.at[t, 0, pl.ds(sid * CPT, CPT)])
      plsc.subcore_barrier()


_bfs_call = pl.kernel(
    _bfs_body,
    out_type=(jax.ShapeDtypeStruct((TT, 1, NPAD), jnp.int32),
              jax.ShapeDtypeStruct((TT, 1, NPAD), jnp.int32)),
    mesh=_mesh,
    compiler_params=_sc_params,
    scratch_types=[
        pltpu.VMEM((EPT1,), jnp.int32),
        pltpu.VMEM((EPT1,), jnp.int32),
        pltpu.VMEM((NPAD,), jnp.int32),
        pltpu.VMEM((NPAD,), jnp.int32),
        pltpu.VMEM((NTILE, CPT), jnp.int32),
        pltpu.VMEM_SHARED((NTILE, 1, NPAD), jnp.int32),
        pltpu.VMEM_SHARED((NPAD,), jnp.int32),
    ],
)


def _msg_body(sidx_hbm, didx_hbm, xwdm_hbm, apart_out,
              sidx_v, didx_v, rows0, rows1, rows2, rows3, zbuf,
              gs0, gs1, gs2, gs3, ss0, ss1, ss2, ss3, a_sh):
  cid = lax.axis_index("c")
  sid = lax.axis_index("s")
  wid = cid * NTILE + sid
  rows = [rows0, rows1, rows2, rows3]
  gsem = [gs0, gs1, gs2, gs3]
  ssem = [ss0, ss1, ss2, ss3]

  def zb(i, _):
    zbuf[i, pl.ds(0, 16)] = jnp.zeros((16,), jnp.float32)
    zbuf[i, pl.ds(16, 16)] = jnp.zeros((16,), jnp.float32)
    zbuf[i, pl.ds(32, 16)] = jnp.zeros((16,), jnp.float32)
    zbuf[i, pl.ds(48, 16)] = jnp.zeros((16,), jnp.float32)
    return 0
  lax.fori_loop(0, CH, zb, 0)

  for t in range(TT):
    for part in range(CPT // CH):
      pltpu.sync_copy(zbuf, a_sh.at[pl.ds(sid * CPT + part * CH, CH)])
    pltpu.sync_copy(sidx_hbm.at[t, wid], sidx_v)
    pltpu.sync_copy(didx_hbm.at[t, wid], didx_v)
    plsc.subcore_barrier()
    gcp = [None] * 4
    scp = [None] * 4
    gcp[0] = pltpu.async_copy(xwdm_hbm.at[sidx_v.at[0]], rows0, gs0)
    gcp[1] = pltpu.async_copy(xwdm_hbm.at[sidx_v.at[1]], rows1, gs1)
    for j in range(NCH):
      b = j % 4
      nj = j + 2
      if nj < NCH:
        nb = nj % 4
        if scp[nb] is not None:
          scp[nb].wait()
          scp[nb] = None
        gcp[nb] = pltpu.async_copy(
            xwdm_hbm.at[sidx_v.at[nj]], rows[nb], gsem[nb])
      gcp[b].wait()
      scp[b] = pltpu.async_copy(rows[b], a_sh.at[didx_v.at[j]], ssem[b],
                                add=True)
    for b in range(4):
      if scp[b] is not None:
        scp[b].wait()
    plsc.subcore_barrier()
    pltpu.sync_copy(a_sh.at[pl.ds(sid * CPT, CPT)],
                    apart_out.at[cid, t, pl.ds(sid * CPT, CPT)])
    plsc.subcore_barrier()


_msg_call = pl.kernel(
    _msg_body,
    out_type=jax.ShapeDtypeStruct((2, TT, NPAD, HH), jnp.float32),
    mesh=_mesh,
    compiler_params=_sc_params,
    scratch_types=[
        pltpu.VMEM((NCH, CH), jnp.int32),
        pltpu.VMEM((NCH, CH), jnp.int32),
        pltpu.VMEM((CH, HH), jnp.float32),
        pltpu.VMEM((CH, HH), jnp.float32),
        pltpu.VMEM((CH, HH), jnp.float32),
        pltpu.VMEM((CH, HH), jnp.float32),
        pltpu.VMEM((CH, HH), jnp.float32),
        pltpu.SemaphoreType.DMA,
        pltpu.SemaphoreType.DMA,
        pltpu.SemaphoreType.DMA,
        pltpu.SemaphoreType.DMA,
        pltpu.SemaphoreType.DMA,
        pltpu.SemaphoreType.DMA,
        pltpu.SemaphoreType.DMA,
        pltpu.SemaphoreType.DMA,
        pltpu.VMEM_SHARED((NPAD, HH), jnp.float32),
    ],
)


def _scal_body(es, ed, gdm_hbm, acc_out,
               s_v, d_v, val_v, acc_v, red_v, stage_sh):
  cid = lax.axis_index("c")
  sid = lax.axis_index("s")

  for t in range(TT):
    @pl.when(cid == t % 2)
    def _(t=t):
      pltpu.sync_copy(es.at[t, sid, 0], s_v)
      pltpu.sync_copy(ed.at[t, sid, 0], d_v)
      pltpu.sync_copy(gdm_hbm.at[t, 0], val_v)
      _zero_vmem(acc_v, NPAD, jnp.float32)

      def ebody(j, _):
        s16 = s_v[pl.ds(j * 16, 16)]
        d16 = d_v[pl.ds(j * 16, 16)]
        gs = plsc.load_gather(val_v, [s16])
        plsc.addupdate_scatter(acc_v, [d16], gs)
        return 0
      lax.fori_loop(0, NV_E, ebody, 0)

      pltpu.sync_copy(acc_v, stage_sh.at[sid, 0])
      plsc.subcore_barrier()
      pltpu.sync_copy(stage_sh.at[:, 0, pl.ds(sid * CPT, CPT)], red_v)
      radd = _reduce_slots(red_v, jnp.float32)

      def rbody(v, _):
        acc_v[pl.ds(sid * CPT + v * 16, 16)] = radd(v, None)
        return 0
      lax.fori_loop(0, NV_C, rbody, 0)
      pltpu.sync_copy(acc_v.at[pl.ds(sid * CPT, CPT)],
                      acc_out.at[t, 0, pl.ds(sid * CPT, CPT)])
      plsc.subcore_barrier()


_scal_call = pl.kernel(
    _scal_body,
    out_type=jax.ShapeDtypeStruct((TT, 1, NPAD), jnp.float32),
    mesh=_mesh,
    compiler_params=_sc_params,
    scratch_types=[
        pltpu.VMEM((EPT1,), jnp.int32),
        pltpu.VMEM((EPT1,), jnp.int32),
        pltpu.VMEM((NPAD,), jnp.float32),
        pltpu.VMEM((NPAD,), jnp.float32),
        pltpu.VMEM((NTILE, CPT), jnp.float32),
        pltpu.VMEM_SHARED((NTILE, 1, NPAD), jnp.float32),
    ],
)


def _win_body(es, ed, nstar_hbm, ccnt_out,
              s_v, d_v, nb_v, cnt_v, red_v, stage_sh):
  cid = lax.axis_index("c")
  sid = lax.axis_index("s")

  for t in range(TT):
    @pl.when(cid == t % 2)
    def _(t=t):
      pltpu.sync_copy(es.at[t, sid, 0], s_v)
      pltpu.sync_copy(ed.at[t, sid, 0], d_v)
      pltpu.sync_copy(nstar_hbm.at[t, 0], nb_v)
      _zero_vmem(cnt_v, NPAD, jnp.int32)
      nst16 = nb_v[...]

      def ebody(j, _):
        s16 = s_v[pl.ds(j * 16, 16)]
        d16 = d_v[pl.ds(j * 16, 16)]
        hit = jnp.where(d16 == nst16, 1, 0)
        plsc.addupdate_scatter(cnt_v, [s16], hit)
        return 0
      lax.fori_loop(0, NV_E, ebody, 0)

      pltpu.sync_copy(cnt_v, stage_sh.at[sid, 0])
      plsc.subcore_barrier()
      pltpu.sync_copy(stage_sh.at[:, 0, pl.ds(sid * CPT, CPT)], red_v)
      radd = _reduce_slots(red_v, jnp.int32)

      def rbody(v, _):
        cnt_v[pl.ds(sid * CPT + v * 16, 16)] = radd(v, None)
        return 0
      lax.fori_loop(0, NV_C, rbody, 0)
      pltpu.sync_copy(cnt_v.at[pl.ds(sid * CPT, CPT)],
                      ccnt_out.at[t, 0, pl.ds(sid * CPT, CPT)])
      plsc.subcore_barrier()


_win_call = pl.kernel(
    _win_body,
    out_type=jax.ShapeDtypeStruct((TT, 1, NPAD), jnp.int32),
    mesh=_mesh,
    compiler_params=_sc_params,
    scratch_types=[
        pltpu.VMEM((EPT1,), jnp.int32),
        pltpu.VMEM((EPT1,), jnp.int32),
        pltpu.VMEM((16,), jnp.int32),
        pltpu.VMEM((NPAD,), jnp.int32),
        pltpu.VMEM((NTILE, CPT), jnp.int32),
        pltpu.VMEM_SHARED((NTILE, 1, NPAD), jnp.int32),
    ],
)


def _prep_body(x_ref, w1a_ref, rc_ref, rnc_ref, tgt_ref, masks_ref, degs_ref,
               xw_out, dinv_out, xwdm_out):
  xw = jnp.dot(x_ref[...], w1a_ref[...], preferred_element_type=jnp.float32)
  ii = lax.broadcasted_iota(jnp.int32, (NN, 1), 0)
  center = (ii == tgt_ref[0]) | (ii == tgt_ref[1])
  xw = xw + jnp.where(center, rc_ref[...], rnc_ref[...])
  xw_out[pl.ds(0, NN), :] = xw
  xw_out[pl.ds(NN, NPAD - NN), :] = jnp.zeros((NPAD - NN, HH), jnp.float32)
  for t in range(TT):
    deg = (degs_ref[pl.ds(t, 1), :] + 1).astype(jnp.float32)
    dinv = lax.rsqrt(deg)
    dinv_out[pl.ds(t, 1), :] = dinv
    dm = dinv * masks_ref[pl.ds(t, 1), :].astype(jnp.float32)
    dmcol = jnp.reshape(dm, (NPAD, 1))
    xwdm_out[pl.ds(t * NPAD, NPAD), :] = xw_out[...] * dmcol


def _prep_call(x, w1a, rc, rnc, tgt, masks, degs):
  return pl.pallas_call(
      _prep_body,
      out_shape=(jax.ShapeDtypeStruct((NPAD, HH), jnp.float32),
                 jax.ShapeDtypeStruct((TT, NPAD), jnp.float32),
                 jax.ShapeDtypeStruct((TT * NPAD, HH), jnp.float32)),
      in_specs=[pl.BlockSpec(memory_space=pltpu.VMEM)] * 4
      + [pl.BlockSpec(memory_space=pltpu.SMEM)]
      + [pl.BlockSpec(memory_space=pltpu.VMEM)] * 2,
  )(x, w1a, rc, rnc, tgt, masks, degs)


def _h1_body(ap_ref, xw_ref, dinv_ref, masks_ref, b1_ref, w2c_ref,
             h1_out, g_out, gdm_out):
  for t in range(TT):
    a = ap_ref[0, pl.ds(t * NPAD, NPAD), :] + ap_ref[1, pl.ds(t * NPAD, NPAD), :]
    dinv = dinv_ref[pl.ds(t, 1), :]
    dcol = jnp.reshape(dinv, (NPAD, 1))
    h1 = jnp.maximum(
        a * dcol + xw_ref[...] * (dcol * dcol) + b1_ref[...], 0.0)
    h1_out[pl.ds(t * NPAD, NPAD), :] = h1
    g = lax.dot_general(w2c_ref[...], h1, (((1,), (1,)), ((), ())),
                        preferred_element_type=jnp.float32)
    g_out[pl.ds(t, 1), :] = g
    gdm_out[pl.ds(t, 1), :] = (
        g * dinv * masks_ref[pl.ds(t, 1), :].astype(jnp.float32))


def _h1_call(ap, xw, dinv, masks, b1r, w2cr):
  return pl.pallas_call(
      _h1_body,
      out_shape=(jax.ShapeDtypeStruct((TT * NPAD, HH), jnp.float32),
                 jax.ShapeDtypeStruct((TT, NPAD), jnp.float32),
                 jax.ShapeDtypeStruct((TT, NPAD), jnp.float32)),
  )(ap, xw, dinv, masks, b1r, w2cr)


def _arg_body(acc_ref, g_ref, dinv_ref, masks_ref, b2h_ref,
              nstar_out, dstar_out):
  for t in range(TT):
    dinv = dinv_ref[pl.ds(t, 1), :]
    tp = jnp.maximum(
        dinv * acc_ref[pl.ds(t, 1), :]
        + g_ref[pl.ds(t, 1), :] * dinv * dinv + b2h_ref[0, 0], 0.0)
    key = jnp.where(masks_ref[pl.ds(t, 1), :] > 0, tp, NEG)
    m = jnp.max(key)
    ii = lax.broadcasted_iota(jnp.int32, (1, NPAD), 1)
    nst = jnp.min(jnp.where(key == m, ii, NPAD))
    dl = jnp.max(jnp.where(ii == nst, dinv, 0.0))
    nstar_out[pl.ds(t, 1), :] = jnp.full((1, 16), nst, jnp.int32)
    dstar_out[pl.ds(t, 1), :] = jnp.full((1, 16), dl, jnp.float32)


def _arg_call(acc, g, dinv, masks, b2h):
  return pl.pallas_call(
      _arg_body,
      out_shape=(jax.ShapeDtypeStruct((TT, 16), jnp.int32),
                 jax.ShapeDtypeStruct((TT, 16), jnp.float32)),
      in_specs=[pl.BlockSpec(memory_space=pltpu.VMEM)] * 4
      + [pl.BlockSpec(memory_space=pltpu.SMEM)],
  )(acc, g, dinv, masks, b2h)


def _final_body(ccnt_ref, dinv_ref, masks_ref, h1_ref, nstar_ref, dstar_ref,
                w2_ref, b2_ref, wir_ref, wiz_ref, win_ref,
                whr_ref, whz_ref, whn_ref, bir_ref, biz_ref, bin_ref,
                bhr_ref, bhz_ref, bhn_ref,
                c1w_ref, c1b_ref, c2w_ref, c2b_ref, out_ref):
  h = jnp.zeros((1, HH), jnp.float32)
  for t in range(TT):
    nst = nstar_ref[t, 0]
    dl = dstar_ref[t, 0]
    crow = (ccnt_ref[pl.ds(t, 1), :].astype(jnp.float32)
            * dinv_ref[pl.ds(t, 1), :]
            * masks_ref[pl.ds(t, 1), :].astype(jnp.float32))
    ii = lax.broadcasted_iota(jnp.int32, (1, NPAD), 1)
    oh = (ii == nst).astype(jnp.float32)
    row = dl * crow + (dl * dl) * oh
    agg = lax.dot_general(row, h1_ref[pl.ds(t * NPAD, NPAD), :],
                          (((1,), (0,)), ((), ())),
                          preferred_element_type=jnp.float32)
    emb = jnp.maximum(
        lax.dot_general(agg, w2_ref[...], (((1,), (0,)), ((), ())),
                        preferred_element_type=jnp.float32) + b2_ref[...], 0.0)
    def mm(v, w_ref, b_ref):
      return lax.dot_general(v, w_ref[...], (((1,), (1,)), ((), ())),
                             preferred_element_type=jnp.float32) + b_ref[...]
    r = jax.nn.sigmoid(mm(emb, wir_ref, bir_ref) + mm(h, whr_ref, bhr_ref))
    z = jax.nn.sigmoid(mm(emb, wiz_ref, biz_ref) + mm(h, whz_ref, bhz_ref))
    nn_ = jnp.tanh(mm(emb, win_ref, bin_ref) + r * mm(h, whn_ref, bhn_ref))
    h = (1.0 - z) * nn_ + z * h
  c = jnp.maximum(
      lax.dot_general(h, c1w_ref[...], (((1,), (1,)), ((), ())),
                      preferred_element_type=jnp.float32) + c1b_ref[...], 0.0)
  score = jax.nn.sigmoid(
      jnp.sum(c * c2w_ref[...], axis=1, keepdims=True) + c2b_ref[...])
  out_ref[...] = score


def _final_call(ccnt, dinv, masks, h1, nstar, dstar, w2, b2r,
                wih, whh, bihr, bhhr, c1w, c1br, c2w, c2br):
  gru = []
  for w in (wih, whh):
    gru += [w[:HH], w[HH:2 * HH], w[2 * HH:]]
  for b in (bihr, bhhr):
    gru += [b[:, :HH], b[:, HH:2 * HH], b[:, 2 * HH:]]
  return pl.pallas_call(
      _final_body,
      out_shape=jax.ShapeDtypeStruct((1, 1), jnp.float32),
      in_specs=[pl.BlockSpec(memory_space=pltpu.VMEM)] * 4
      + [pl.BlockSpec(memory_space=pltpu.SMEM)] * 2
      + [pl.BlockSpec(memory_space=pltpu.VMEM)] * 18,
  )(ccnt, dinv, masks, h1, nstar, dstar, w2, b2r,
    *gru, c1w, c1br, c2w, c2br)


@jax.jit
def kernel(x, edge_index, target_nodes, W1, b1, W2, b2,
           Wih, Whh, bih, bhh, C1w, C1b, C2w, C2b):
  ei = edge_index.astype(jnp.int32)
  tgt = target_nodes.astype(jnp.int32)
  initmask = jnp.zeros((NPAD,), jnp.int32).at[tgt].set(1)

  pad1 = EP1 - EE
  es = jnp.pad(ei[:, 0, :], ((0, 0), (0, pad1)),
               constant_values=NPAD - 1).reshape(TT, NTILE, 1, EPT1)
  ed = jnp.pad(ei[:, 1, :], ((0, 0), (0, pad1)),
               constant_values=NPAD - 1).reshape(TT, NTILE, 1, EPT1)

  masks3, degs3 = _bfs_call(es, ed, initmask)
  masks = masks3.reshape(TT, NPAD)
  degs = degs3.reshape(TT, NPAD)

  w1a = W1[:DD]
  rc = W1[DD].reshape(1, HH)
  rnc = W1[DD + 1].reshape(1, HH)
  xw, dinv, xwdm = _prep_call(x, w1a, rc, rnc, tgt, masks, degs)

  pad = EPAD - EE
  spad = jnp.pad(ei[:, 0, :], ((0, 0), (0, pad)), constant_values=NPAD - 1)
  dpad = jnp.pad(ei[:, 1, :], ((0, 0), (0, pad)), constant_values=NPAD - 1)
  toff = (jnp.arange(TT, dtype=jnp.int32) * NPAD)[:, None, None, None]
  sidx = spad.reshape(TT, 32, NCH, CH) + toff
  didx = dpad.reshape(TT, 32, NCH, CH)
  ap = _msg_call(sidx, didx, xwdm)

  h1, g, gdm = _h1_call(ap.reshape(2, TT * NPAD, HH), xw, dinv, masks,
                        b1.reshape(1, HH), W2[:, -1].reshape(1, HH))
  acc = _scal_call(es, ed, gdm.reshape(TT, 1, NPAD)).reshape(TT, NPAD)
  nstar, dstar = _arg_call(acc, g, dinv, masks, b2[-1].reshape(1, 1))
  ccnt = _win_call(es, ed, nstar.reshape(TT, 1, 16)).reshape(TT, NPAD)
  out = _final_call(ccnt, dinv, masks, h1, nstar, dstar, W2,
                    b2.reshape(1, HH), Wih, Whh, bih.reshape(1, 3 * HH),
                    bhh.reshape(1, 3 * HH), C1w, C1b.reshape(1, HH // 2),
                    C2w, C2b.reshape(1, 1))
  return out.reshape(())


# speedup vs baseline: 73.7036x; 1.0006x over previous
import jax
import jax.numpy as jnp
from jax import lax
from jax.experimental import pallas as pl
from jax.experimental.pallas import tpu as pltpu
from jax.experimental.pallas import tpu_sc as plsc

NN = 10000
EE = 320000
TT = 3
DD = 128
HH = 64
NPAD = 10240
NTILE = 16
CPT = NPAD // NTILE
NV_C = CPT // 16
EPT1 = 20480
NV_E = EPT1 // 16
EP1 = NTILE * EPT1
CH = 128
EPT2 = 10112
NCH = EPT2 // CH
EPAD = 32 * EPT2
NEG = -1e30

_mesh = plsc.VectorSubcoreMesh(core_axis_name="c", subcore_axis_name="s")
_sc_params = pltpu.CompilerParams(needs_layout_passes=False,
                                  use_tc_tiling_on_sc=False)


def _zero_vmem(ref, n, dtype):
  def body(i, _):
    ref[pl.ds(i * 16, 16)] = jnp.zeros((16,), dtype)
    return 0
  lax.fori_loop(0, n // 16, body, 0)


def _reduce_slots(red_v, dtype):
  def body(v, _):
    acc = red_v[0, pl.ds(v * 16, 16)]
    for k in range(1, NTILE):
      acc = acc + red_v[k, pl.ds(v * 16, 16)]
    return acc
  return body


def _bfs_body(es, ed, initmask, masks_out, degs_out,
              s_v, d_v, mask_v, cnt_v, red_v, stage_sh, mask_sh):
  cid = lax.axis_index("c")
  sid = lax.axis_index("s")

  for t in range(TT):
    @pl.when(cid == t % 2)
    def _(t=t):
      pltpu.sync_copy(es.at[t, sid, 0], s_v)
      pltpu.sync_copy(ed.at[t, sid, 0], d_v)
      pltpu.sync_copy(initmask, mask_v)
      for _hop in range(2):
        _zero_vmem(cnt_v, NPAD, jnp.int32)

        def ebody(j, _):
          s16 = s_v[pl.ds(j * 16, 16)]
          d16 = d_v[pl.ds(j * 16, 16)]
          ms = plsc.load_gather(mask_v, [s16])
          md = plsc.load_gather(mask_v, [d16])
          plsc.addupdate_scatter(cnt_v, [d16], ms)
          plsc.addupdate_scatter(cnt_v, [s16], md)
          return 0
        lax.fori_loop(0, NV_E, ebody, 0)

        pltpu.sync_copy(cnt_v, stage_sh.at[sid, 0])
        plsc.subcore_barrier()
        pltpu.sync_copy(stage_sh.at[:, 0, pl.ds(sid * CPT, CPT)], red_v)
        radd = _reduce_slots(red_v, jnp.int32)

        def rbody(v, _):
          acc = radd(v, None)
          old = mask_v[pl.ds(sid * CPT + v * 16, 16)]
          mask_v[pl.ds(sid * CPT + v * 16, 16)] = jnp.where(acc > 0, 1, old)
          return 0
        lax.fori_loop(0, NV_C, rbody, 0)

        pltpu.sync_copy(mask_v.at[pl.ds(sid * CPT, CPT)],
                        mask_sh.at[pl.ds(sid * CPT, CPT)])
        plsc.subcore_barrier()
        pltpu.sync_copy(mask_sh, mask_v)
        plsc.subcore_barrier()

      pltpu.sync_copy(mask_v.at[pl.ds(sid * CPT, CPT)],
                      masks_out.at[t, 0, pl.ds(sid * CPT, CPT)])

      _zero_vmem(cnt_v, NPAD, jnp.int32)

      def dbody(j, _):
        s16 = s_v[pl.ds(j * 16, 16)]
        d16 = d_v[pl.ds(j * 16, 16)]
        ms = plsc.load_gather(mask_v, [s16])
        plsc.addupdate_scatter(cnt_v, [d16], ms)
        return 0
      lax.fori_loop(0, NV_E, dbody, 0)

      pltpu.sync_copy(cnt_v, stage_sh.at[sid, 0])
      plsc.subcore_barrier()
      pltpu.sync_copy(stage_sh.at[:, 0, pl.ds(sid * CPT, CPT)], red_v)
      radd = _reduce_slots(red_v, jnp.int32)

      def dbody2(v, _):
        cnt_v[pl.ds(sid * CPT + v * 16, 16)] = radd(v, None)
        return 0
      lax.fori_loop(0, NV_C, dbody2, 0)
      pltpu.sync_copy(cnt_v.at[pl.ds(sid * CPT, CPT)],
                      degs_out.at[t, 0, pl.ds(sid * CPT, CPT)])
      plsc.subcore_barrier()


_bfs_call = pl.kernel(
    _bfs_body,
    out_type=(jax.ShapeDtypeStruct((TT, 1, NPAD), jnp.int32),
              jax.ShapeDtypeStruct((TT, 1, NPAD), jnp.int32)),
    mesh=_mesh,
    compiler_params=_sc_params,
    scratch_types=[
        pltpu.VMEM((EPT1,), jnp.int32),
        pltpu.VMEM((EPT1,), jnp.int32),
        pltpu.VMEM((NPAD,), jnp.int32),
        pltpu.VMEM((NPAD,), jnp.int32),
        pltpu.VMEM((NTILE, CPT), jnp.int32),
        pltpu.VMEM_SHARED((NTILE, 1, NPAD), jnp.int32),
        pltpu.VMEM_SHARED((NPAD,), jnp.int32),
    ],
)


def _msg_body(sidx_hbm, didx_hbm, xwdm_hbm, apart_out,
              sidx_v, didx_v, rows0, rows1, rows2, rows3, zbuf,
              gs0, gs1, gs2, gs3, ss0, ss1, ss2, ss3, a_sh):
  cid = lax.axis_index("c")
  sid = lax.axis_index("s")
  wid = cid * NTILE + sid
  rows = [rows0, rows1, rows2, rows3]
  gsem = [gs0, gs1, gs2, gs3]
  ssem = [ss0, ss1, ss2, ss3]

  def zb(i, _):
    zbuf[i, pl.ds(0, 16)] = jnp.zeros((16,), jnp.float32)
    zbuf[i, pl.ds(16, 16)] = jnp.zeros((16,), jnp.float32)
    zbuf[i, pl.ds(32, 16)] = jnp.zeros((16,), jnp.float32)
    zbuf[i, pl.ds(48, 16)] = jnp.zeros((16,), jnp.float32)
    return 0
  lax.fori_loop(0, CH, zb, 0)

  for t in range(TT):
    for part in range(CPT // CH):
      pltpu.sync_copy(zbuf, a_sh.at[pl.ds(sid * CPT + part * CH, CH)])
    pltpu.sync_copy(sidx_hbm.at[t, wid], sidx_v)
    pltpu.sync_copy(didx_hbm.at[t, wid], didx_v)
    plsc.subcore_barrier()
    gcp = [None] * 4
    scp = [None] * 4
    gcp[0] = pltpu.async_copy(xwdm_hbm.at[sidx_v.at[0]], rows0, gs0)
    gcp[1] = pltpu.async_copy(xwdm_hbm.at[sidx_v.at[1]], rows1, gs1)
    for j in range(NCH):
      b = j % 4
      nj = j + 2
      if nj < NCH:
        nb = nj % 4
        if scp[nb] is not None:
          scp[nb].wait()
          scp[nb] = None
        gcp[nb] = pltpu.async_copy(
            xwdm_hbm.at[sidx_v.at[nj]], rows[nb], gsem[nb])
      gcp[b].wait()
      scp[b] = pltpu.async_copy(rows[b], a_sh.at[didx_v.at[j]], ssem[b],
                                add=True)
    for b in range(4):
      if scp[b] is not None:
        scp[b].wait()
    plsc.subcore_barrier()
    pltpu.sync_copy(a_sh.at[pl.ds(sid * CPT, CPT)],
                    apart_out.at[cid, t, pl.ds(sid * CPT, CPT)])
    plsc.subcore_barrier()


_msg_call = pl.kernel(
    _msg_body,
    out_type=jax.ShapeDtypeStruct((2, TT, NPAD, HH), jnp.float32),
    mesh=_mesh,
    compiler_params=_sc_params,
    scratch_types=[
        pltpu.VMEM((NCH, CH), jnp.int32),
        pltpu.VMEM((NCH, CH), jnp.int32),
        pltpu.VMEM((CH, HH), jnp.float32),
        pltpu.VMEM((CH, HH), jnp.float32),
        pltpu.VMEM((CH, HH), jnp.float32),
        pltpu.VMEM((CH, HH), jnp.float32),
        pltpu.VMEM((CH, HH), jnp.float32),
        pltpu.SemaphoreType.DMA,
        pltpu.SemaphoreType.DMA,
        pltpu.SemaphoreType.DMA,
        pltpu.SemaphoreType.DMA,
        pltpu.SemaphoreType.DMA,
        pltpu.SemaphoreType.DMA,
        pltpu.SemaphoreType.DMA,
        pltpu.SemaphoreType.DMA,
        pltpu.VMEM_SHARED((NPAD, HH), jnp.float32),
    ],
)


def _scal_body(es, ed, gdm_hbm, acc_out,
               s_v, d_v, val_v, acc_v, red_v, stage_sh):
  cid = lax.axis_index("c")
  sid = lax.axis_index("s")

  for t in range(TT):
    @pl.when(cid == t % 2)
    def _(t=t):
      pltpu.sync_copy(es.at[t, sid, 0], s_v)
      pltpu.sync_copy(ed.at[t, sid, 0], d_v)
      pltpu.sync_copy(gdm_hbm.at[t, 0], val_v)
      _zero_vmem(acc_v, NPAD, jnp.float32)

      def ebody(j, _):
        s16 = s_v[pl.ds(j * 16, 16)]
        d16 = d_v[pl.ds(j * 16, 16)]
        gs = plsc.load_gather(val_v, [s16])
        plsc.addupdate_scatter(acc_v, [d16], gs)
        return 0
      lax.fori_loop(0, NV_E, ebody, 0)

      pltpu.sync_copy(acc_v, stage_sh.at[sid, 0])
      plsc.subcore_barrier()
      pltpu.sync_copy(stage_sh.at[:, 0, pl.ds(sid * CPT, CPT)], red_v)
      radd = _reduce_slots(red_v, jnp.float32)

      def rbody(v, _):
        acc_v[pl.ds(sid * CPT + v * 16, 16)] = radd(v, None)
        return 0
      lax.fori_loop(0, NV_C, rbody, 0)
      pltpu.sync_copy(acc_v.at[pl.ds(sid * CPT, CPT)],
                      acc_out.at[t, 0, pl.ds(sid * CPT, CPT)])
      plsc.subcore_barrier()


_scal_call = pl.kernel(
    _scal_body,
    out_type=jax.ShapeDtypeStruct((TT, 1, NPAD), jnp.float32),
    mesh=_mesh,
    compiler_params=_sc_params,
    scratch_types=[
        pltpu.VMEM((EPT1,), jnp.int32),
        pltpu.VMEM((EPT1,), jnp.int32),
        pltpu.VMEM((NPAD,), jnp.float32),
        pltpu.VMEM((NPAD,), jnp.float32),
        pltpu.VMEM((NTILE, CPT), jnp.float32),
        pltpu.VMEM_SHARED((NTILE, 1, NPAD), jnp.float32),
    ],
)


def _win_body(es, ed, nstar_hbm, ccnt_out,
              s_v, d_v, nb_v, cnt_v, red_v, stage_sh):
  cid = lax.axis_index("c")
  sid = lax.axis_index("s")

  for t in range(TT):
    @pl.when(cid == t % 2)
    def _(t=t):
      pltpu.sync_copy(es.at[t, sid, 0], s_v)
      pltpu.sync_copy(ed.at[t, sid, 0], d_v)
      pltpu.sync_copy(nstar_hbm.at[t, 0], nb_v)
      _zero_vmem(cnt_v, NPAD, jnp.int32)
      nst16 = nb_v[...]

      def ebody(j, _):
        s16 = s_v[pl.ds(j * 16, 16)]
        d16 = d_v[pl.ds(j * 16, 16)]
        hit = jnp.where(d16 == nst16, 1, 0)
        plsc.addupdate_scatter(cnt_v, [s16], hit)
        return 0
      lax.fori_loop(0, NV_E, ebody, 0)

      pltpu.sync_copy(cnt_v, stage_sh.at[sid, 0])
      plsc.subcore_barrier()
      pltpu.sync_copy(stage_sh.at[:, 0, pl.ds(sid * CPT, CPT)], red_v)
      radd = _reduce_slots(red_v, jnp.int32)

      def rbody(v, _):
        cnt_v[pl.ds(sid * CPT + v * 16, 16)] = radd(v, None)
        return 0
      lax.fori_loop(0, NV_C, rbody, 0)
      pltpu.sync_copy(cnt_v.at[pl.ds(sid * CPT, CPT)],
                      ccnt_out.at[t, 0, pl.ds(sid * CPT, CPT)])
      plsc.subcore_barrier()


_win_call = pl.kernel(
    _win_body,
    out_type=jax.ShapeDtypeStruct((TT, 1, NPAD), jnp.int32),
    mesh=_mesh,
    compiler_params=_sc_params,
    scratch_types=[
        pltpu.VMEM((EPT1,), jnp.int32),
        pltpu.VMEM((EPT1,), jnp.int32),
        pltpu.VMEM((16,), jnp.int32),
        pltpu.VMEM((NPAD,), jnp.int32),
        pltpu.VMEM((NTILE, CPT), jnp.int32),
        pltpu.VMEM_SHARED((NTILE, 1, NPAD), jnp.int32),
    ],
)


def _prep_body(x_ref, w1a_ref, rc_ref, rnc_ref, tgt_ref, masks_ref, degs_ref,
               xw_out, dinv_out, xwdm_out):
  xw = jnp.dot(x_ref[...], w1a_ref[...], preferred_element_type=jnp.float32)
  ii = lax.broadcasted_iota(jnp.int32, (NN, 1), 0)
  center = (ii == tgt_ref[0]) | (ii == tgt_ref[1])
  xw = xw + jnp.where(center, rc_ref[...], rnc_ref[...])
  xw_out[pl.ds(0, NN), :] = xw
  xw_out[pl.ds(NN, NPAD - NN), :] = jnp.zeros((NPAD - NN, HH), jnp.float32)
  for t in range(TT):
    deg = (degs_ref[pl.ds(t, 1), :] + 1).astype(jnp.float32)
    dinv = lax.rsqrt(deg)
    dinv_out[pl.ds(t, 1), :] = dinv
    dm = dinv * masks_ref[pl.ds(t, 1), :].astype(jnp.float32)
    dmcol = jnp.reshape(dm, (NPAD, 1))
    xwdm_out[pl.ds(t * NPAD, NPAD), :] = xw_out[...] * dmcol


def _prep_call(x, w1a, rc, rnc, tgt, masks, degs):
  return pl.pallas_call(
      _prep_body,
      out_shape=(jax.ShapeDtypeStruct((NPAD, HH), jnp.float32),
                 jax.ShapeDtypeStruct((TT, NPAD), jnp.float32),
                 jax.ShapeDtypeStruct((TT * NPAD, HH), jnp.float32)),
      in_specs=[pl.BlockSpec(memory_space=pltpu.VMEM)] * 4
      + [pl.BlockSpec(memory_space=pltpu.SMEM)]
      + [pl.BlockSpec(memory_space=pltpu.VMEM)] * 2,
  )(x, w1a, rc, rnc, tgt, masks, degs)


def _h1_body(ap_ref, xw_ref, dinv_ref, masks_ref, b1_ref, w2c_ref,
             h1_out, g_out, gdm_out):
  for t in range(TT):
    a = ap_ref[0, pl.ds(t * NPAD, NPAD), :] + ap_ref[1, pl.ds(t * NPAD, NPAD), :]
    dinv = dinv_ref[pl.ds(t, 1), :]
    dcol = jnp.reshape(dinv, (NPAD, 1))
    h1 = jnp.maximum(
        a * dcol + xw_ref[...] * (dcol * dcol) + b1_ref[...], 0.0)
    h1_out[pl.ds(t * NPAD, NPAD), :] = h1
    g = lax.dot_general(w2c_ref[...], h1, (((1,), (1,)), ((), ())),
                        preferred_element_type=jnp.float32)
    g_out[pl.ds(t, 1), :] = g
    gdm_out[pl.ds(t, 1), :] = (
        g * dinv * masks_ref[pl.ds(t, 1), :].astype(jnp.float32))


def _h1_call(ap, xw, dinv, masks, b1r, w2cr):
  return pl.pallas_call(
      _h1_body,
      out_shape=(jax.ShapeDtypeStruct((TT * NPAD, HH), jnp.float32),
                 jax.ShapeDtypeStruct((TT, NPAD), jnp.float32),
                 jax.ShapeDtypeStruct((TT, NPAD), jnp.float32)),
  )(ap, xw, dinv, masks, b1r, w2cr)


def _arg_body(acc_ref, g_ref, dinv_ref, masks_ref, b2h_ref,
              nstar_out, dstar_out):
  for t in range(TT):
    dinv = dinv_ref[pl.ds(t, 1), :]
    tp = jnp.maximum(
        dinv * acc_ref[pl.ds(t, 1), :]
        + g_ref[pl.ds(t, 1), :] * dinv * dinv + b2h_ref[0, 0], 0.0)
    key = jnp.where(masks_ref[pl.ds(t, 1), :] > 0, tp, NEG)
    m = jnp.max(key)
    ii = lax.broadcasted_iota(jnp.int32, (1, NPAD), 1)
    nst = jnp.min(jnp.where(key == m, ii, NPAD))
    dl = jnp.max(jnp.where(ii == nst, dinv, 0.0))
    nstar_out[pl.ds(t, 1), :] = jnp.full((1, 16), nst, jnp.int32)
    dstar_out[pl.ds(t, 1), :] = jnp.full((1, 16), dl, jnp.float32)


def _arg_call(acc, g, dinv, masks, b2h):
  return pl.pallas_call(
      _arg_body,
      out_shape=(jax.ShapeDtypeStruct((TT, 16), jnp.int32),
                 jax.ShapeDtypeStruct((TT, 16), jnp.float32)),
      in_specs=[pl.BlockSpec(memory_space=pltpu.VMEM)] * 4
      + [pl.BlockSpec(memory_space=pltpu.SMEM)],
  )(acc, g, dinv, masks, b2h)


def _final_body(ccnt_ref, dinv_ref, masks_ref, h1_ref, nstar_ref, dstar_ref,
                w2_ref, b2_ref, wir_ref, wiz_ref, win_ref,
                whr_ref, whz_ref, whn_ref, bir_ref, biz_ref, bin_ref,
                bhr_ref, bhz_ref, bhn_ref,
                c1w_ref, c1b_ref, c2w_ref, c2b_ref, out_ref):
  h = jnp.zeros((1, HH), jnp.float32)
  for t in range(TT):
    nst = nstar_ref[t, 0]
    dl = dstar_ref[t, 0]
    crow = (ccnt_ref[pl.ds(t, 1), :].astype(jnp.float32)
            * dinv_ref[pl.ds(t, 1), :]
            * masks_ref[pl.ds(t, 1), :].astype(jnp.float32))
    ii = lax.broadcasted_iota(jnp.int32, (1, NPAD), 1)
    oh = (ii == nst).astype(jnp.float32)
    row = dl * crow + (dl * dl) * oh
    agg = lax.dot_general(row, h1_ref[pl.ds(t * NPAD, NPAD), :],
                          (((1,), (0,)), ((), ())),
                          preferred_element_type=jnp.float32)
    emb = jnp.maximum(
        lax.dot_general(agg, w2_ref[...], (((1,), (0,)), ((), ())),
                        preferred_element_type=jnp.float32) + b2_ref[...], 0.0)
    def mm(v, w_ref, b_ref):
      return lax.dot_general(v, w_ref[...], (((1,), (1,)), ((), ())),
                             preferred_element_type=jnp.float32) + b_ref[...]
    r = jax.nn.sigmoid(mm(emb, wir_ref, bir_ref) + mm(h, whr_ref, bhr_ref))
    z = jax.nn.sigmoid(mm(emb, wiz_ref, biz_ref) + mm(h, whz_ref, bhz_ref))
    nn_ = jnp.tanh(mm(emb, win_ref, bin_ref) + r * mm(h, whn_ref, bhn_ref))
    h = (1.0 - z) * nn_ + z * h
  c = jnp.maximum(
      lax.dot_general(h, c1w_ref[...], (((1,), (1,)), ((), ())),
                      preferred_element_type=jnp.float32) + c1b_ref[...], 0.0)
  score = jax.nn.sigmoid(
      jnp.sum(c * c2w_ref[...], axis=1, keepdims=True) + c2b_ref[...])
  out_ref[...] = score


def _final_call(ccnt, dinv, masks, h1, nstar, dstar, w2, b2r,
                wih, whh, bihr, bhhr, c1w, c1br, c2w, c2br):
  gru = []
  for w in (wih, whh):
    gru += [w[:HH], w[HH:2 * HH], w[2 * HH:]]
  for b in (bihr, bhhr):
    gru += [b[:, :HH], b[:, HH:2 * HH], b[:, 2 * HH:]]
  return pl.pallas_call(
      _final_body,
      out_shape=jax.ShapeDtypeStruct((1, 1), jnp.float32),
      in_specs=[pl.BlockSpec(memory_space=pltpu.VMEM)] * 4
      + [pl.BlockSpec(memory_space=pltpu.SMEM)] * 2
      + [pl.BlockSpec(memory_space=pltpu.VMEM)] * 18,
  )(ccnt, dinv, masks, h1, nstar, dstar, w2, b2r,
    *gru, c1w, c1br, c2w, c2br)


@jax.jit
def kernel(x, edge_index, target_nodes, W1, b1, W2, b2,
           Wih, Whh, bih, bhh, C1w, C1b, C2w, C2b):
  ei = edge_index.astype(jnp.int32)
  tgt = target_nodes.astype(jnp.int32)
  initmask = jnp.zeros((NPAD,), jnp.int32).at[tgt].set(1)

  pad1 = EP1 - EE
  es = jnp.pad(ei[:, 0, :], ((0, 0), (0, pad1)),
               constant_values=NPAD - 1).reshape(TT, NTILE, 1, EPT1)
  ed = jnp.pad(ei[:, 1, :], ((0, 0), (0, pad1)),
               constant_values=NPAD - 1).reshape(TT, NTILE, 1, EPT1)

  masks3, degs3 = _bfs_call(es, ed, initmask)
  masks = masks3.reshape(TT, NPAD)
  degs = degs3.reshape(TT, NPAD)

  w1a = W1[:DD]
  rc = W1[DD].reshape(1, HH)
  rnc = W1[DD + 1].reshape(1, HH)
  xw, dinv, xwdm = _prep_call(x, w1a, rc, rnc, tgt, masks, degs)

  pad = EPAD - EE
  spad = jnp.pad(ei[:, 0, :], ((0, 0), (0, pad)), constant_values=NPAD - 1)
  dpad = jnp.pad(ei[:, 1, :], ((0, 0), (0, pad)), constant_values=NPAD - 1)
  toff = (jnp.arange(TT, dtype=jnp.int32) * NPAD)[:, None, None, None]
  sidx = spad.reshape(TT, 32, NCH, CH) + toff
  didx = dpad.reshape(TT, 32, NCH, CH)
  ap = _msg_call(sidx, didx, xwdm)

  h1, g, gdm = _h1_call(ap.reshape(2, TT * NPAD, HH), xw, dinv, masks,
                        b1.reshape(1, HH), W2[:, -1].reshape(1, HH))
  acc = _scal_call(es, ed, gdm.reshape(TT, 1, NPAD)).reshape(TT, NPAD)
  nstar, dstar = _arg_call(acc, g, dinv, masks, b2[-1].reshape(1, 1))
  ccnt = _win_call(es, ed, nstar.reshape(TT, 1, 16)).reshape(TT, NPAD)
  out = _final_call(ccnt, dinv, masks, h1, nstar, dstar, W2,
                    b2.reshape(1, HH), Wih, Whh, bih.reshape(1, 3 * HH),
                    bhh.reshape(1, 3 * HH), C1w, C1b.reshape(1, HH // 2),
                    C2w, C2b.reshape(1, 1))
  return out.reshape(())
